# baseline (speedup 1.0000x reference)
import dataclasses
import functools

import jax
import jax.numpy as jnp
from jax import lax
from jax.experimental import pallas as pl
from jax.experimental.pallas import tpu as pltpu
from jax.experimental.pallas import tpu_sc as plsc

NUM_ROWS = 17
DIM = 10
ROW_PAD = 24
B = 16384
LEN = 200
LT = LEN // 8

NC, NS, L = 2, 16, 16
NW = NC * NS
WB = B // NW
NT = WB // 128

_MESH = plsc.VectorSubcoreMesh(core_axis_name="c", subcore_axis_name="s")

_CP = pltpu.CompilerParams()
if "needs_layout_passes" in pltpu.CompilerParams.__dataclass_fields__:
    _CP = dataclasses.replace(_CP, needs_layout_passes=False)
_CP = dataclasses.replace(_CP, use_tc_tiling_on_sc=True)


@functools.partial(
    pl.kernel,
    out_type=jax.ShapeDtypeStruct((DIM, LEN, B), jnp.float32),
    mesh=_MESH,
    compiler_params=_CP,
    scratch_types=[
        pltpu.VMEM((DIM * ROW_PAD,), jnp.float32),
        pltpu.VMEM((8, WB), jnp.int32),
        pltpu.VMEM((DIM, 8, WB), jnp.float32),
        pltpu.SemaphoreType.DMA,
    ],
)
def _sc_lookup(tab_hbm, idx_hbm, out_hbm, tab_v, idx_v, out_v, sem):
    wid = lax.axis_index("s") * NC + lax.axis_index("c")
    b0 = wid * WB
    pltpu.sync_copy(tab_hbm, tab_v)

    @pl.loop(0, LT)
    def _(lt):
        pltpu.sync_copy(idx_hbm.at[pl.ds(lt * 8, 8), pl.ds(b0, WB)], idx_v)

        @pl.loop(0, 8)
        def _(i):
            @pl.loop(0, WB // L)
            def _(j):
                idxv = idx_v[i, pl.ds(j * L, L)]
                for d in range(DIM):
                    vals = plsc.load_gather(
                        tab_v.at[pl.ds(d * ROW_PAD, ROW_PAD)], [idxv]
                    )
                    out_v[d, i, pl.ds(j * L, L)] = vals

        copies = [
            pltpu.async_copy(
                out_v.at[d], out_hbm.at[d, pl.ds(lt * 8, 8), pl.ds(b0, WB)], sem
            )
            for d in range(DIM)
        ]
        for c in copies:
            c.wait()


def kernel(list_POSs, table):
    idx_t = list_POSs.astype(jnp.int32).T
    tab_dm = jnp.pad(
        table.astype(jnp.float32).T, ((0, 0), (0, ROW_PAD - NUM_ROWS))
    )
    out_t = _sc_lookup(tab_dm.reshape(-1), idx_t)
    return jnp.transpose(out_t, (2, 1, 0))


# speedup vs baseline: 3.4852x; 3.4852x over previous
import jax
import jax.numpy as jnp
from jax.experimental import pallas as pl
from jax.experimental.pallas import tpu as pltpu

NUM_ROWS = 17
DIM = 10
B = 16384
LEN = 200

BLK_B = 4096


def _lookup_body(tab_ref, idx_ref, out_ref):
    idxb = idx_ref[...]
    ilo = idxb & 7
    ihi = (idxb - 8) & 7
    is_lo = idxb < 8
    is_16 = idxb == 16
    for d in range(DIM):
        a = jnp.take_along_axis(tab_ref[d, 0:8], ilo, axis=0,
                                mode="promise_in_bounds")
        bv = jnp.take_along_axis(tab_ref[d, 8:16], ihi, axis=0,
                                 mode="promise_in_bounds")
        r = jnp.where(is_lo, a, bv)
        out_ref[d, :, :] = jnp.where(is_16, tab_ref[d, 16], r)


@jax.jit
def _lookup(idx_t, tab_lanes):
    return pl.pallas_call(
        _lookup_body,
        out_shape=jax.ShapeDtypeStruct((DIM, LEN, B), jnp.float32),
        grid=(LEN // 8, B // BLK_B),
        in_specs=[
            pl.BlockSpec((DIM, NUM_ROWS, BLK_B), lambda i, j: (0, 0, 0)),
            pl.BlockSpec((8, BLK_B), lambda i, j: (i, j)),
        ],
        out_specs=pl.BlockSpec((DIM, 8, BLK_B), lambda i, j: (0, i, j)),
        compiler_params=pltpu.CompilerParams(
            dimension_semantics=("parallel", "parallel")
        ),
    )(tab_lanes, idx_t)


def kernel(list_POSs, table):
    idx_t = list_POSs.astype(jnp.int32).T
    tab_lanes = jnp.broadcast_to(
        table.astype(jnp.float32).T[:, :, None], (DIM, NUM_ROWS, BLK_B)
    )
    out_t = _lookup(idx_t, tab_lanes)
    return jnp.transpose(out_t, (2, 1, 0))


# speedup vs baseline: 5.9102x; 1.6958x over previous
import jax
import jax.numpy as jnp
from jax.experimental import pallas as pl
from jax.experimental.pallas import tpu as pltpu

NUM_ROWS = 17
DIM = 10
B = 16384
LEN = 200

BLK_B = 512
GRID = B // BLK_B


def _lookup_body(tab_ref, idx_ref, out_ref):
    idxb = idx_ref[...]
    ilo = idxb & 7
    ihi = (idxb - 8) & 7
    is_lo = idxb < 8
    is_16 = idxb == 16
    for d in range(DIM):
        a = jnp.take_along_axis(tab_ref[d, 0:8], ilo, axis=0,
                                mode="promise_in_bounds")
        bv = jnp.take_along_axis(tab_ref[d, 8:16], ihi, axis=0,
                                 mode="promise_in_bounds")
        r = jnp.where(is_lo, a, bv)
        out_ref[d, :, :] = jnp.where(is_16, tab_ref[d, 16], r)


@jax.jit
def _lookup(idx_t, tab_lanes):
    return pl.pallas_call(
        _lookup_body,
        out_shape=jax.ShapeDtypeStruct((DIM, LEN, B), jnp.float32),
        grid=(GRID,),
        in_specs=[
            pl.BlockSpec((DIM, NUM_ROWS, BLK_B), lambda i: (0, 0, 0)),
            pl.BlockSpec((LEN, BLK_B), lambda i: (0, i)),
        ],
        out_specs=pl.BlockSpec((DIM, LEN, BLK_B), lambda i: (0, 0, i)),
        compiler_params=pltpu.CompilerParams(
            dimension_semantics=("parallel",)
        ),
    )(tab_lanes, idx_t)


def kernel(list_POSs, table):
    idx_t = list_POSs.astype(jnp.int32).T
    tab_lanes = jnp.broadcast_to(
        table.astype(jnp.float32).T[:, :, None], (DIM, NUM_ROWS, BLK_B)
    )
    out_t = _lookup(idx_t, tab_lanes)
    return jnp.transpose(out_t, (2, 1, 0))


# speedup vs baseline: 6.7459x; 1.1414x over previous
import jax
import jax.numpy as jnp
from jax.experimental import pallas as pl
from jax.experimental.pallas import tpu as pltpu

NUM_ROWS = 17
DIM = 10
B = 16384
LEN = 200

BLK_B = 1024
GRID = B // BLK_B


def _lookup_body(tab_ref, idx_ref, out_ref):
    idxb = idx_ref[...]
    ilo = idxb & 7
    ihi = (idxb - 8) & 7
    is_lo = idxb < 8
    is_16 = idxb == 16
    for d in range(DIM):
        a = jnp.take_along_axis(tab_ref[d, 0:8], ilo, axis=0,
                                mode="promise_in_bounds")
        bv = jnp.take_along_axis(tab_ref[d, 8:16], ihi, axis=0,
                                 mode="promise_in_bounds")
        r = jnp.where(is_lo, a, bv)
        out_ref[d, :, :] = jnp.where(is_16, tab_ref[d, 16], r)


@jax.jit
def _lookup(idx_t, tab_lanes):
    return pl.pallas_call(
        _lookup_body,
        out_shape=jax.ShapeDtypeStruct((DIM, LEN, B), jnp.float32),
        grid=(GRID,),
        in_specs=[
            pl.BlockSpec((DIM, NUM_ROWS, BLK_B), lambda i: (0, 0, 0)),
            pl.BlockSpec((LEN, BLK_B), lambda i: (0, i)),
        ],
        out_specs=pl.BlockSpec((DIM, LEN, BLK_B), lambda i: (0, 0, i)),
        compiler_params=pltpu.CompilerParams(
            dimension_semantics=("parallel",)
        ),
    )(tab_lanes, idx_t)


def kernel(list_POSs, table):
    idx_t = list_POSs.astype(jnp.int32).T
    tab_lanes = jnp.broadcast_to(
        table.astype(jnp.float32).T[:, :, None], (DIM, NUM_ROWS, BLK_B)
    )
    out_t = _lookup(idx_t, tab_lanes)
    return jnp.transpose(out_t, (2, 1, 0))
